# baseline (speedup 1.0000x reference)
import functools

import jax
import jax.numpy as jnp
from jax import lax
from jax.experimental import pallas as pl
from jax.experimental.pallas import tpu as pltpu
from jax.experimental.pallas import tpu_sc as plsc

N_NODES = 10000
D = 256
E = 160000
NC = 2
NS = 16
NW = NC * NS
EC = 64
EPW = 5120
EP = EPW * NW
NCHUNK = EPW // EC
NG = EC // 16
U = 8


def _body(re_hbm, irh_hbm, row_hbm, col_hbm, out_hbm,
          row_v, col_v, rr_v, rc_v, hr_v, hc_v, out_v, sem):
    cid = lax.axis_index("c")
    sid = lax.axis_index("s")
    wid = sid * NC + cid
    base = wid * EPW
    pltpu.sync_copy(row_hbm.at[pl.ds(base, EPW)], row_v)
    pltpu.sync_copy(col_hbm.at[pl.ds(base, EPW)], col_v)
    iota = lax.broadcasted_iota(jnp.int32, (16,), 0)
    zf = jnp.zeros((16,), jnp.float32)
    zi = jnp.zeros((16,), jnp.int32)

    def chunk_body(c, acc):
        off = c * EC
        cp1 = pltpu.async_copy(re_hbm.at[row_v.at[pl.ds(off, EC)]], rr_v, sem)
        cp2 = pltpu.async_copy(re_hbm.at[col_v.at[pl.ds(off, EC)]], rc_v, sem)
        cp3 = pltpu.async_copy(irh_hbm.at[row_v.at[pl.ds(off, EC)]], hr_v, sem)
        cp4 = pltpu.async_copy(irh_hbm.at[col_v.at[pl.ds(off, EC)]], hc_v, sem)
        cp1.wait()
        cp2.wait()
        cp3.wait()
        cp4.wait()
        for g in range(NG):
            rows16 = iota + (g * 16)

            def feat_body(f, carry):
                dot0, dot1, dif0, dif1 = carry
                for k in range(U):
                    fk = jnp.broadcast_to(f + k, (16,))
                    ar = plsc.load_gather(rr_v, [rows16, fk])
                    ac = plsc.load_gather(rc_v, [rows16, fk])
                    hr = plsc.load_gather(hr_v, [rows16, fk])
                    hc = plsc.load_gather(hc_v, [rows16, fk])
                    d = hr - hc
                    if k % 2 == 0:
                        dot0 = dot0 + ar * ac
                        dif0 = dif0 + d * d
                    else:
                        dot1 = dot1 + ar * ac
                        dif1 = dif1 + d * d
                return dot0, dot1, dif0, dif1

            dot0, dot1, dif0, dif1 = plsc.parallel_loop(
                0, D, step=U, unroll=2, carry=(zf, zf, zf, zf))(feat_body)
            dotv = dot0 + dot1
            difv = dif0 + dif1
            dis = 1.0 / (1.0 + jnp.exp(dotv))
            acc = acc + dis * difv
        return acc

    acc = lax.fori_loop(0, NCHUNK, chunk_body, zf)
    out_v[...] = acc
    pltpu.sync_copy(out_v, out_hbm.at[wid])


_sc_call = functools.partial(
    pl.kernel,
    out_type=jax.ShapeDtypeStruct((NW, 16), jnp.float32),
    mesh=plsc.VectorSubcoreMesh(core_axis_name="c", subcore_axis_name="s"),
    compiler_params=pltpu.CompilerParams(
        use_tc_tiling_on_sc=False, needs_layout_passes=False),
    scratch_types=[
        pltpu.VMEM((EPW,), jnp.int32),
        pltpu.VMEM((EPW,), jnp.int32),
        pltpu.VMEM((EC, D), jnp.float32),
        pltpu.VMEM((EC, D), jnp.float32),
        pltpu.VMEM((EC, D), jnp.float32),
        pltpu.VMEM((EC, D), jnp.float32),
        pltpu.VMEM((16,), jnp.float32),
        pltpu.SemaphoreType.DMA,
    ],
)(_body)


def kernel(re_, ir_h, edge_index):
    row = jnp.pad(edge_index[0], (0, EP - E))
    col = jnp.pad(edge_index[1], (0, EP - E))
    partials = _sc_call(re_, ir_h, row, col)
    return jnp.sum(partials) / E


# speedup vs baseline: 3.0466x; 3.0466x over previous
import functools

import jax
import jax.numpy as jnp
from jax import lax
from jax.experimental import pallas as pl
from jax.experimental.pallas import tpu as pltpu
from jax.experimental.pallas import tpu_sc as plsc

N_NODES = 10000
D = 256
E = 160000
NC = 2
NS = 16
NW = NC * NS
EC = 64
EPW = 5120
EP = EPW * NW
NCHUNK = EPW // EC
NG = EC // 16
U = 8


def _body(re_hbm, irh_hbm, row_hbm, col_hbm, out_hbm,
          row_v, col_v, rr_v, rc_v, hr_v, hc_v, dots_v, difs_v, out_v, sem):
    cid = lax.axis_index("c")
    sid = lax.axis_index("s")
    wid = sid * NC + cid
    base = wid * EPW
    pltpu.sync_copy(row_hbm.at[pl.ds(base, EPW)], row_v)
    pltpu.sync_copy(col_hbm.at[pl.ds(base, EPW)], col_v)
    iota = lax.broadcasted_iota(jnp.int32, (16,), 0)
    zf = jnp.zeros((16,), jnp.float32)
    zi = jnp.zeros((16,), jnp.int32)

    def chunk_body(c, acc):
        off = c * EC
        cp1 = pltpu.async_copy(re_hbm.at[row_v.at[pl.ds(off, EC)]], rr_v, sem)
        cp2 = pltpu.async_copy(re_hbm.at[col_v.at[pl.ds(off, EC)]], rc_v, sem)
        cp3 = pltpu.async_copy(irh_hbm.at[row_v.at[pl.ds(off, EC)]], hr_v, sem)
        cp4 = pltpu.async_copy(irh_hbm.at[col_v.at[pl.ds(off, EC)]], hc_v, sem)
        cp1.wait()
        cp2.wait()
        cp3.wait()
        cp4.wait()

        def edge_body(e):
            dot0 = zf
            dot1 = zf
            dif0 = zf
            dif1 = zf
            for k in range(D // 16):
                sl = pl.ds(k * 16, 16)
                ar = rr_v[e, sl]
                ac = rc_v[e, sl]
                hr = hr_v[e, sl]
                hc = hc_v[e, sl]
                d = hr - hc
                if k % 2 == 0:
                    dot0 = dot0 + ar * ac
                    dif0 = dif0 + d * d
                else:
                    dot1 = dot1 + ar * ac
                    dif1 = dif1 + d * d
            dots_v[e, pl.ds(0, 16)] = dot0 + dot1
            difs_v[e, pl.ds(0, 16)] = dif0 + dif1

        plsc.parallel_loop(0, EC, step=1, unroll=2)(edge_body)

        for g in range(NG):
            rows16 = iota + (g * 16)
            dotv = zf
            difv = zf
            for l in range(16):
                l16 = jnp.full((16,), l, jnp.int32)
                dotv = dotv + plsc.load_gather(dots_v, [rows16, l16])
                difv = difv + plsc.load_gather(difs_v, [rows16, l16])
            dis = 1.0 / (1.0 + jnp.exp(dotv))
            acc = acc + dis * difv
        return acc

    acc = lax.fori_loop(0, NCHUNK, chunk_body, zf)
    out_v[...] = acc
    pltpu.sync_copy(out_v, out_hbm.at[wid])


_sc_call = functools.partial(
    pl.kernel,
    out_type=jax.ShapeDtypeStruct((NW, 16), jnp.float32),
    mesh=plsc.VectorSubcoreMesh(core_axis_name="c", subcore_axis_name="s"),
    compiler_params=pltpu.CompilerParams(
        use_tc_tiling_on_sc=False, needs_layout_passes=False),
    scratch_types=[
        pltpu.VMEM((EPW,), jnp.int32),
        pltpu.VMEM((EPW,), jnp.int32),
        pltpu.VMEM((EC, D), jnp.float32),
        pltpu.VMEM((EC, D), jnp.float32),
        pltpu.VMEM((EC, D), jnp.float32),
        pltpu.VMEM((EC, D), jnp.float32),
        pltpu.VMEM((EC, 17), jnp.float32),
        pltpu.VMEM((EC, 17), jnp.float32),
        pltpu.VMEM((16,), jnp.float32),
        pltpu.SemaphoreType.DMA,
    ],
)(_body)


def kernel(re_, ir_h, edge_index):
    row = jnp.pad(edge_index[0], (0, EP - E))
    col = jnp.pad(edge_index[1], (0, EP - E))
    partials = _sc_call(re_, ir_h, row, col)
    return jnp.sum(partials) / E


# speedup vs baseline: 3.6017x; 1.1822x over previous
import functools

import jax
import jax.numpy as jnp
from jax import lax
from jax.experimental import pallas as pl
from jax.experimental.pallas import tpu as pltpu
from jax.experimental.pallas import tpu_sc as plsc

N_NODES = 10000
D = 256
E = 160000
NC = 2
NS = 16
NW = NC * NS
EC = 64
EPW = 5120
EP = EPW * NW
NCHUNK = EPW // EC
NG = EC // 16
D2 = D // 2


def _body(re_hbm, irh_hbm, row_hbm, col_hbm, out_hbm,
          row_v, col_v, rr_v, rc_v, hr_v, hc_v, dots_v, difs_v, out_v, sem):
    cid = lax.axis_index("c")
    sid = lax.axis_index("s")
    wid = sid * NC + cid
    base = wid * EPW
    pltpu.sync_copy(row_hbm.at[pl.ds(base, EPW)], row_v)
    pltpu.sync_copy(col_hbm.at[pl.ds(base, EPW)], col_v)
    iota = lax.broadcasted_iota(jnp.int32, (16,), 0)
    zf = jnp.zeros((16,), jnp.float32)
    zi = jnp.zeros((16,), jnp.int32)

    def chunk_body(c, acc):
        off = c * EC
        cp1 = pltpu.async_copy(re_hbm.at[row_v.at[pl.ds(off, EC)]], rr_v, sem)
        cp2 = pltpu.async_copy(re_hbm.at[col_v.at[pl.ds(off, EC)]], rc_v, sem)
        cp3 = pltpu.async_copy(irh_hbm.at[row_v.at[pl.ds(off, EC)]], hr_v, sem)
        cp4 = pltpu.async_copy(irh_hbm.at[col_v.at[pl.ds(off, EC)]], hc_v, sem)
        cp1.wait()
        cp2.wait()
        cp3.wait()
        cp4.wait()

        zb = jnp.zeros((32,), jnp.bfloat16)

        def edge_body(e):
            dot0 = zf
            dot1 = zf
            dif0 = zb
            dif1 = zb
            for k in range(D2 // 16):
                sl = pl.ds(k * 16, 16)
                ar = plsc.bitcast(rr_v[e, sl], jnp.bfloat16)
                ac = plsc.bitcast(rc_v[e, sl], jnp.bfloat16)
                hr = plsc.bitcast(hr_v[e, sl], jnp.bfloat16)
                hc = plsc.bitcast(hc_v[e, sl], jnp.bfloat16)
                ae, ao = plsc.unpack(ar, format=plsc.PackFormat.INTERLEAVED)
                ce, co = plsc.unpack(ac, format=plsc.PackFormat.INTERLEAVED)
                dot0 = dot0 + ae * ce
                dot1 = dot1 + ao * co
                d = hr - hc
                if k % 2 == 0:
                    dif0 = dif0 + d * d
                else:
                    dif1 = dif1 + d * d
            de, do = plsc.unpack(dif0 + dif1, format=plsc.PackFormat.INTERLEAVED)
            dots_v[e, pl.ds(0, 16)] = dot0 + dot1
            difs_v[e, pl.ds(0, 16)] = de + do

        plsc.parallel_loop(0, EC, step=1, unroll=2)(edge_body)

        for g in range(NG):
            rows16 = iota + (g * 16)
            dotv = zf
            difv = zf
            for l in range(16):
                l16 = jnp.full((16,), l, jnp.int32)
                dotv = dotv + plsc.load_gather(dots_v, [rows16, l16])
                difv = difv + plsc.load_gather(difs_v, [rows16, l16])
            dis = 1.0 / (1.0 + jnp.exp(dotv))
            acc = acc + dis * difv
        return acc

    acc = lax.fori_loop(0, NCHUNK, chunk_body, zf)
    out_v[...] = acc
    pltpu.sync_copy(out_v, out_hbm.at[wid])


_sc_call = functools.partial(
    pl.kernel,
    out_type=jax.ShapeDtypeStruct((NW, 16), jnp.float32),
    mesh=plsc.VectorSubcoreMesh(core_axis_name="c", subcore_axis_name="s"),
    compiler_params=pltpu.CompilerParams(
        use_tc_tiling_on_sc=False, needs_layout_passes=False),
    scratch_types=[
        pltpu.VMEM((EPW,), jnp.int32),
        pltpu.VMEM((EPW,), jnp.int32),
        pltpu.VMEM((EC, D2), jnp.int32),
        pltpu.VMEM((EC, D2), jnp.int32),
        pltpu.VMEM((EC, D2), jnp.int32),
        pltpu.VMEM((EC, D2), jnp.int32),
        pltpu.VMEM((EC, 17), jnp.float32),
        pltpu.VMEM((EC, 17), jnp.float32),
        pltpu.VMEM((16,), jnp.float32),
        pltpu.SemaphoreType.DMA,
    ],
)(_body)


def kernel(re_, ir_h, edge_index):
    row = jnp.pad(edge_index[0], (0, EP - E))
    col = jnp.pad(edge_index[1], (0, EP - E))
    re_b = jax.lax.bitcast_convert_type(
        re_.astype(jnp.bfloat16).reshape(N_NODES, D2, 2), jnp.int32)
    irh_b = jax.lax.bitcast_convert_type(
        ir_h.astype(jnp.bfloat16).reshape(N_NODES, D2, 2), jnp.int32)
    partials = _sc_call(re_b, irh_b, row, col)
    return jnp.sum(partials) / E


# speedup vs baseline: 4.2644x; 1.1840x over previous
import functools

import jax
import jax.numpy as jnp
from jax import lax
from jax.experimental import pallas as pl
from jax.experimental.pallas import tpu as pltpu
from jax.experimental.pallas import tpu_sc as plsc

N_NODES = 10000
D = 256
E = 160000
NC = 2
NS = 16
NW = NC * NS
EC = 64
EPW = 5120
EP = EPW * NW
NCHUNK = EPW // EC
NG = EC // 16
D2 = D // 2


def _body(re_hbm, irh_hbm, row_hbm, col_hbm, out_hbm,
          row_v, col_v,
          rr0_v, rc0_v, hr0_v, hc0_v,
          rr1_v, rc1_v, hr1_v, hc1_v,
          dots_v, difs_v, out_v, sem0, sem1):
    cid = lax.axis_index("c")
    sid = lax.axis_index("s")
    wid = sid * NC + cid
    base = wid * EPW
    pltpu.sync_copy(row_hbm.at[pl.ds(base, EPW)], row_v)
    pltpu.sync_copy(col_hbm.at[pl.ds(base, EPW)], col_v)
    iota = lax.broadcasted_iota(jnp.int32, (16,), 0)
    zf = jnp.zeros((16,), jnp.float32)
    zb = jnp.zeros((32,), jnp.bfloat16)
    bufs = ((rr0_v, rc0_v, hr0_v, hc0_v, sem0),
            (rr1_v, rc1_v, hr1_v, hc1_v, sem1))

    def issue(c, bset):
        rr, rc, hr, hc, sem = bset
        off = c * EC
        r_idx = row_v.at[pl.ds(off, EC)]
        c_idx = col_v.at[pl.ds(off, EC)]
        pltpu.async_copy(re_hbm.at[r_idx], rr, sem)
        pltpu.async_copy(re_hbm.at[c_idx], rc, sem)
        pltpu.async_copy(irh_hbm.at[r_idx], hr, sem)
        pltpu.async_copy(irh_hbm.at[c_idx], hc, sem)

    def drain(bset):
        rr, rc, hr, hc, sem = bset
        z_idx = row_v.at[pl.ds(0, EC)]
        pltpu.make_async_copy(re_hbm.at[z_idx], rr, sem).wait()
        pltpu.make_async_copy(re_hbm.at[z_idx], rc, sem).wait()
        pltpu.make_async_copy(irh_hbm.at[z_idx], hr, sem).wait()
        pltpu.make_async_copy(irh_hbm.at[z_idx], hc, sem).wait()

    def compute(bset, acc):
        rr_v, rc_v, hr_v, hc_v, _ = bset

        def edge_body(e):
            dot0 = zf
            dot1 = zf
            dif0 = zb
            dif1 = zb
            for k in range(D2 // 16):
                sl = pl.ds(k * 16, 16)
                ar = plsc.bitcast(rr_v[e, sl], jnp.bfloat16)
                ac = plsc.bitcast(rc_v[e, sl], jnp.bfloat16)
                hr = plsc.bitcast(hr_v[e, sl], jnp.bfloat16)
                hc = plsc.bitcast(hc_v[e, sl], jnp.bfloat16)
                ae, ao = plsc.unpack(ar, format=plsc.PackFormat.INTERLEAVED)
                ce, co = plsc.unpack(ac, format=plsc.PackFormat.INTERLEAVED)
                dot0 = dot0 + ae * ce
                dot1 = dot1 + ao * co
                d = hr - hc
                if k % 2 == 0:
                    dif0 = dif0 + d * d
                else:
                    dif1 = dif1 + d * d
            de, do = plsc.unpack(dif0 + dif1, format=plsc.PackFormat.INTERLEAVED)
            dots_v[e, pl.ds(0, 16)] = dot0 + dot1
            difs_v[e, pl.ds(0, 16)] = de + do

        plsc.parallel_loop(0, EC, step=1, unroll=2)(edge_body)

        for g in range(NG):
            rows16 = iota + (g * 16)
            dotv = zf
            difv = zf
            for l in range(16):
                l16 = jnp.full((16,), l, jnp.int32)
                dotv = dotv + plsc.load_gather(dots_v, [rows16, l16])
                difv = difv + plsc.load_gather(difs_v, [rows16, l16])
            dis = 1.0 / (1.0 + jnp.exp(dotv))
            acc = acc + dis * difv
        return acc

    issue(0, bufs[0])
    issue(1, bufs[1])

    def pair_body(p, acc):
        c = p * 2
        drain(bufs[0])
        acc = compute(bufs[0], acc)
        issue(c + 2, bufs[0])
        drain(bufs[1])
        acc = compute(bufs[1], acc)
        issue(c + 3, bufs[1])
        return acc

    acc = lax.fori_loop(0, NCHUNK // 2 - 1, pair_body, zf)
    drain(bufs[0])
    acc = compute(bufs[0], acc)
    drain(bufs[1])
    acc = compute(bufs[1], acc)
    out_v[...] = acc
    pltpu.sync_copy(out_v, out_hbm.at[wid])


_sc_call = functools.partial(
    pl.kernel,
    out_type=jax.ShapeDtypeStruct((NW, 16), jnp.float32),
    mesh=plsc.VectorSubcoreMesh(core_axis_name="c", subcore_axis_name="s"),
    compiler_params=pltpu.CompilerParams(
        use_tc_tiling_on_sc=False, needs_layout_passes=False),
    scratch_types=[
        pltpu.VMEM((EPW,), jnp.int32),
        pltpu.VMEM((EPW,), jnp.int32),
        pltpu.VMEM((EC, D2), jnp.int32),
        pltpu.VMEM((EC, D2), jnp.int32),
        pltpu.VMEM((EC, D2), jnp.int32),
        pltpu.VMEM((EC, D2), jnp.int32),
        pltpu.VMEM((EC, D2), jnp.int32),
        pltpu.VMEM((EC, D2), jnp.int32),
        pltpu.VMEM((EC, D2), jnp.int32),
        pltpu.VMEM((EC, D2), jnp.int32),
        pltpu.VMEM((EC, 17), jnp.float32),
        pltpu.VMEM((EC, 17), jnp.float32),
        pltpu.VMEM((16,), jnp.float32),
        pltpu.SemaphoreType.DMA,
        pltpu.SemaphoreType.DMA,
    ],
)(_body)


def kernel(re_, ir_h, edge_index):
    row = jnp.pad(edge_index[0], (0, EP - E))
    col = jnp.pad(edge_index[1], (0, EP - E))
    re_b = jax.lax.bitcast_convert_type(
        re_.astype(jnp.bfloat16).reshape(N_NODES, D2, 2), jnp.int32)
    irh_b = jax.lax.bitcast_convert_type(
        ir_h.astype(jnp.bfloat16).reshape(N_NODES, D2, 2), jnp.int32)
    partials = _sc_call(re_b, irh_b, row, col)
    return jnp.sum(partials) / E
